# baseline (speedup 1.0000x reference)
import functools

import jax
import jax.numpy as jnp
from jax import lax
from jax.experimental import pallas as pl
from jax.experimental.pallas import tpu as pltpu
from jax.experimental.pallas import tpu_sc as plsc

_V = 8192
_D = 64
_B = 1048576
_C = 8192
_NCH = _B // _C
_CB = _C // 128
_NB = 3


def _make_gather():
    info = plsc.get_sparse_core_info()
    mesh = plsc.VectorSubcoreMesh(core_axis_name="c", subcore_axis_name="s")

    @functools.partial(
        pl.kernel,
        mesh=mesh,
        out_type=jax.ShapeDtypeStruct((8, _B // 128, 8, 128), jnp.float32),
        scratch_types=[
            pltpu.VMEM((2, _V), jnp.float32),
            pltpu.VMEM((_NB, _C), jnp.int32),
            pltpu.VMEM((_NB, 2, _CB, 128), jnp.float32),
            pltpu.VMEM_SHARED((_NB, _C), jnp.int32),
            [pltpu.SemaphoreType.DMA] * _NB,
            [pltpu.SemaphoreType.DMA] * _NB,
            [pltpu.SemaphoreType.DMA] * _NB,
        ],
        compiler_params=pltpu.CompilerParams(
            use_tc_tiling_on_sc=False, needs_layout_passes=False),
    )
    def gather_kernel(idx_hbm, ct_hbm, out_hbm, cols, idxb, outb, sidx,
                      hsems, lsems, wsems):
        sid = lax.axis_index("s")
        wid = sid * info.num_cores + lax.axis_index("c")
        d0 = wid * 2
        a = d0 // 8
        c = d0 % 8

        pltpu.sync_copy(ct_hbm.at[pl.ds(d0, 2)], cols)

        def start_hbm_idx(q, b):
            @pl.when(sid == 0)
            def _():
                off = pl.multiple_of(q * _C, 8)
                pltpu.async_copy(idx_hbm.at[pl.ds(off, _C)], sidx.at[b],
                                 hsems[b])

        def wait_hbm_idx(q, b):
            @pl.when(sid == 0)
            def _():
                off = pl.multiple_of(q * _C, 8)
                pltpu.make_async_copy(idx_hbm.at[pl.ds(off, _C)], sidx.at[b],
                                      hsems[b]).wait()

        def start_local_idx(b):
            pltpu.async_copy(sidx.at[b], idxb.at[b], lsems[b])

        def wait_local_idx(b):
            pltpu.make_async_copy(sidx.at[b], idxb.at[b], lsems[b]).wait()

        def compute(b):
            @plsc.parallel_loop(0, _CB, unroll=8)
            def bb_body(bb):
                for u in range(8):
                    ivec = idxb[b, pl.ds(bb * 128 + u * 16, 16)]
                    for dl in range(2):
                        vals = plsc.load_gather(cols.at[dl], [ivec])
                        outb[b, dl, bb, pl.ds(u * 16, 16)] = vals

        def start_write(q, b):
            for dl in range(2):
                pltpu.async_copy(
                    outb.at[b, dl], out_hbm.at[a, pl.ds(q * _CB, _CB), c + dl],
                    wsems[b])

        def wait_write(q, b):
            for dl in range(2):
                pltpu.make_async_copy(
                    outb.at[b, dl], out_hbm.at[a, pl.ds(q * _CB, _CB), c + dl],
                    wsems[b]).wait()

        start_hbm_idx(0, 0)

        def body(p, carry):
            for bpar in range(_NB):
                pp = p * _NB + bpar
                bc = (bpar - 1) % _NB
                q = pp - 1

                @pl.when(jnp.logical_and(q >= 0, q < _NCH))
                def _():
                    wait_local_idx(bc)

                @pl.when(pp < _NCH)
                def _():
                    wait_hbm_idx(pp, bpar)

                plsc.subcore_barrier()

                @pl.when(pp + 1 < _NCH)
                def _():
                    start_hbm_idx(pp + 1, (bpar + 1) % _NB)

                @pl.when(pp < _NCH)
                def _():
                    start_local_idx(bpar)

                @pl.when(jnp.logical_and(q >= 0, q < _NCH))
                def _():
                    @pl.when(q >= _NB)
                    def _():
                        wait_write(q - _NB, bc)

                    compute(bc)
                    start_write(q, bc)
            return carry

        lax.fori_loop(0, (_NCH + _NB) // _NB, body, 0)

        for q in range(_NCH - _NB, _NCH):
            wait_write(q, q % _NB)

    return gather_kernel


_gather = _make_gather()


def kernel(indexes, codebook):
    idx = indexes.astype(jnp.int32)
    ct = codebook.T
    out4 = _gather(idx, ct)
    return out4.transpose(1, 3, 0, 2).reshape(_B, _D)


# speedup vs baseline: 1.0285x; 1.0285x over previous
import functools

import jax
import jax.numpy as jnp
from jax import lax
from jax.experimental import pallas as pl
from jax.experimental.pallas import tpu as pltpu
from jax.experimental.pallas import tpu_sc as plsc

_V = 8192
_D = 64
_B = 1048576
_C = 16384
_NCH = _B // _C
_CB = _C // 128
_NB = 2


def _make_gather():
    info = plsc.get_sparse_core_info()
    mesh = plsc.VectorSubcoreMesh(core_axis_name="c", subcore_axis_name="s")

    @functools.partial(
        pl.kernel,
        mesh=mesh,
        out_type=jax.ShapeDtypeStruct((8, _B // 128, 8, 128), jnp.float32),
        scratch_types=[
            pltpu.VMEM((2, _V), jnp.float32),
            pltpu.VMEM((_NB, _C), jnp.int32),
            pltpu.VMEM((_NB, _CB, 2, 128), jnp.float32),
            pltpu.VMEM_SHARED((_NB, _C), jnp.int32),
            [pltpu.SemaphoreType.DMA] * _NB,
            [pltpu.SemaphoreType.DMA] * _NB,
            [pltpu.SemaphoreType.DMA] * _NB,
        ],
        compiler_params=pltpu.CompilerParams(
            use_tc_tiling_on_sc=False, needs_layout_passes=False),
    )
    def gather_kernel(idx_hbm, ct_hbm, out_hbm, cols, idxb, outb, sidx,
                      hsems, lsems, wsems):
        sid = lax.axis_index("s")
        wid = sid * info.num_cores + lax.axis_index("c")
        d0 = wid * 2
        a = d0 // 8
        c = d0 % 8

        pltpu.sync_copy(ct_hbm.at[pl.ds(d0, 2)], cols)

        def start_hbm_idx(q, b):
            @pl.when(sid == 0)
            def _():
                off = pl.multiple_of(q * _C, 8)
                pltpu.async_copy(idx_hbm.at[pl.ds(off, _C)], sidx.at[b],
                                 hsems[b])

        def wait_hbm_idx(q, b):
            @pl.when(sid == 0)
            def _():
                off = pl.multiple_of(q * _C, 8)
                pltpu.make_async_copy(idx_hbm.at[pl.ds(off, _C)], sidx.at[b],
                                      hsems[b]).wait()

        def start_local_idx(b):
            pltpu.async_copy(sidx.at[b], idxb.at[b], lsems[b])

        def wait_local_idx(b):
            pltpu.make_async_copy(sidx.at[b], idxb.at[b], lsems[b]).wait()

        def compute(b):
            @plsc.parallel_loop(0, _CB, unroll=8)
            def bb_body(bb):
                for u in range(8):
                    ivec = idxb[b, pl.ds(bb * 128 + u * 16, 16)]
                    for dl in range(2):
                        vals = plsc.load_gather(cols.at[dl], [ivec])
                        outb[b, bb, dl, pl.ds(u * 16, 16)] = vals

        def start_write(q, b):
            pltpu.async_copy(
                outb.at[b], out_hbm.at[a, pl.ds(q * _CB, _CB), pl.ds(c, 2)],
                wsems[b])

        def wait_write(q, b):
            pltpu.make_async_copy(
                outb.at[b], out_hbm.at[a, pl.ds(q * _CB, _CB), pl.ds(c, 2)],
                wsems[b]).wait()

        start_hbm_idx(0, 0)

        def body(p, carry):
            for bpar in range(_NB):
                pp = p * _NB + bpar
                bc = bpar ^ 1
                q = pp - 1

                @pl.when(jnp.logical_and(q >= 0, q < _NCH))
                def _():
                    wait_local_idx(bc)

                @pl.when(pp < _NCH)
                def _():
                    wait_hbm_idx(pp, bpar)

                plsc.subcore_barrier()

                @pl.when(pp + 1 < _NCH)
                def _():
                    start_hbm_idx(pp + 1, bc)

                @pl.when(pp < _NCH)
                def _():
                    start_local_idx(bpar)

                @pl.when(jnp.logical_and(q >= 0, q < _NCH))
                def _():
                    @pl.when(q >= 2)
                    def _():
                        wait_write(q - 2, bc)

                    compute(bc)
                    start_write(q, bc)
            return carry

        lax.fori_loop(0, (_NCH + 2) // _NB, body, 0)

        for q in (_NCH - 2, _NCH - 1):
            wait_write(q, q % _NB)

    return gather_kernel


_gather = _make_gather()


def kernel(indexes, codebook):
    idx = indexes.astype(jnp.int32)
    ct = codebook.T
    out4 = _gather(idx, ct)
    return out4.transpose(1, 3, 0, 2).reshape(_B, _D)
